# Initial kernel scaffold; baseline (speedup 1.0000x reference)
import jax
import jax.numpy as jnp
from jax.experimental import pallas as pl


def kernel(x, W_in, W_lin, W_out):
    raise NotImplementedError("write your pallas kernel here")


# speedup vs baseline: 1.3051x; 1.3051x over previous
import functools

import jax
import jax.numpy as jnp
from jax.experimental import pallas as pl
from jax.experimental.pallas import tpu as pltpu

N, D, H, E, K = 32768, 768, 64, 64, 2
CHUNK = 2048
GRID = N // CHUNK


def _gate_body(x_ref, w_ref, win_ref, wlin_ref, idx_ref, p_ref, acc_ref):
    i = pl.program_id(0)

    @pl.when(i == 0)
    def _init():
        acc_ref[...] = jnp.zeros_like(acc_ref)

    w = w_ref[...]
    xb = x_ref[...]
    acc_ref[...] += jax.lax.dot_general(
        w, xb, (((1,), (0,)), ((), ())),
        preferred_element_type=jnp.float32)

    @pl.when(i == GRID - 1)
    def _finish():
        v = acc_ref[...]
        h = jax.lax.dot_general(
            v, win_ref[...], (((1,), (1,)), ((), ())),
            preferred_element_type=jnp.float32)
        o = jax.lax.dot_general(
            h, wlin_ref[...], (((1,), (1,)), ((), ())),
            preferred_element_type=jnp.float32)

        iota = jax.lax.broadcasted_iota(jnp.int32, (1, E), 1)
        m1 = jnp.max(o)
        i1 = jnp.min(jnp.where(o == m1, iota, E))
        masked = jnp.where(iota == i1, -jnp.inf, o)
        m2 = jnp.max(masked)
        i2 = jnp.min(jnp.where(masked == m2, iota, E))
        e = jnp.exp(m2 - m1)
        p1 = 1.0 / (1.0 + e)

        pos = jax.lax.broadcasted_iota(jnp.int32, (1, 2), 1)
        idx_ref[...] = jnp.where(pos == 0, i1, i2)
        p_ref[...] = jnp.where(pos == 0, p1, 1.0 - p1)


@functools.partial(jax.jit, static_argnames=("interpret",))
def kernel(x, W_in, W_lin, W_out, interpret=False):
    idx2, p2 = pl.pallas_call(
        _gate_body,
        grid=(GRID,),
        in_specs=[
            pl.BlockSpec((CHUNK, D), lambda i: (i, 0)),
            pl.BlockSpec((1, CHUNK), lambda i: (0, i)),
            pl.BlockSpec((H, D), lambda i: (0, 0)),
            pl.BlockSpec((E, H), lambda i: (0, 0)),
        ],
        out_specs=[
            pl.BlockSpec((1, 2), lambda i: (0, 0)),
            pl.BlockSpec((1, 2), lambda i: (0, 0)),
        ],
        out_shape=[
            jax.ShapeDtypeStruct((1, 2), jnp.int32),
            jax.ShapeDtypeStruct((1, 2), jnp.float32),
        ],
        scratch_shapes=[pltpu.VMEM((1, D), jnp.float32)],
        interpret=interpret,
    )(x, W_out, W_in, W_lin)
    return idx2.reshape(-1), p2.reshape(-1)
